# baseline (speedup 1.0000x reference)
import functools

import jax
import jax.numpy as jnp
from jax import lax
from jax.experimental import pallas as pl
from jax.experimental.pallas import tpu as pltpu
from jax.experimental.pallas import tpu_sc as plsc

BM = 128
GATE_BM = 512
_NC = 2
_NS = 16
_NW = _NC * _NS


def _sc_mesh():
    return plsc.VectorSubcoreMesh(core_axis_name="c", subcore_axis_name="s")


def _sc_gather(x, row_token, p):
    row_shape = x.shape[1:]
    row_elems = 1
    for s in row_shape:
        row_elems *= s
    row_bytes = row_elems * x.dtype.itemsize
    rpw = p // _NW
    ch = max(8, (128 * 1024) // row_bytes)
    while rpw % ch:
        ch //= 2
    nbuf = 3
    nch = rpw // ch

    @functools.partial(
        pl.kernel,
        out_type=jax.ShapeDtypeStruct((p,) + row_shape, x.dtype),
        mesh=_sc_mesh(),
        scratch_types=[
            pltpu.VMEM((rpw,), jnp.int32),
            [pltpu.VMEM((ch,) + row_shape, x.dtype)] * nbuf,
            [pltpu.SemaphoreType.DMA] * nbuf,
            [pltpu.SemaphoreType.DMA] * nbuf,
        ],
    )
    def k(tok_hbm, x_hbm, xs_hbm, idx_v, bufs, gsems, wsems):
        wid = lax.axis_index("s") * _NC + lax.axis_index("c")
        base = wid * rpw
        pltpu.sync_copy(tok_hbm.at[pl.ds(base, rpw)], idx_v)

        def g(c):
            return pltpu.make_async_copy(
                x_hbm.at[idx_v.at[pl.ds(c * ch, ch)]], bufs[c % nbuf],
                gsems[c % nbuf])

        def w(c):
            return pltpu.make_async_copy(
                bufs[c % nbuf], xs_hbm.at[pl.ds(base + c * ch, ch)],
                wsems[c % nbuf])

        g(0).start()
        g(1).start()
        for c in range(nch):
            g(c).wait()
            w(c).start()
            if c + 2 < nch:
                if c >= 1:
                    w(c - 1).wait()
                g(c + 2).start()
        for c in range(max(nch - 3, 0), nch):
            w(c).wait()

    return k(row_token, x)


def _sc_combine(zs, pos0, pos1):
    b = pos0.shape[0]
    zdim = zs.shape[1]
    tpw = b // _NW
    ch = 64
    nch = tpw // ch

    @functools.partial(
        pl.kernel,
        out_type=jax.ShapeDtypeStruct((b, zdim), jnp.float32),
        mesh=_sc_mesh(),
        scratch_types=[
            pltpu.VMEM((tpw,), jnp.int32),
            pltpu.VMEM((tpw,), jnp.int32),
            pltpu.VMEM((ch, zdim), jnp.float32),
            pltpu.VMEM((ch, zdim), jnp.float32),
            pltpu.VMEM((ch, zdim), jnp.float32),
            pltpu.VMEM((ch, zdim), jnp.float32),
            pltpu.SemaphoreType.DMA,
            pltpu.SemaphoreType.DMA,
            pltpu.SemaphoreType.DMA,
            pltpu.SemaphoreType.DMA,
            pltpu.SemaphoreType.DMA,
            pltpu.SemaphoreType.DMA,
        ],
    )
    def k(zs_hbm, p0_hbm, p1_hbm, out_hbm, i0_v, i1_v,
          a0, a1, c0, c1, ga0, ga1, gb0, gb1, w0, w1):
        wid = lax.axis_index("s") * _NC + lax.axis_index("c")
        base = wid * tpw
        pltpu.sync_copy(p0_hbm.at[pl.ds(base, tpw)], i0_v)
        pltpu.sync_copy(p1_hbm.at[pl.ds(base, tpw)], i1_v)
        abufs, cbufs = (a0, a1), (c0, c1)
        gasems, gbsems, wsems = (ga0, ga1), (gb0, gb1), (w0, w1)

        def ga(c):
            return pltpu.make_async_copy(
                zs_hbm.at[i0_v.at[pl.ds(c * ch, ch)]], abufs[c % 2],
                gasems[c % 2])

        def gb(c):
            return pltpu.make_async_copy(
                zs_hbm.at[i1_v.at[pl.ds(c * ch, ch)]], cbufs[c % 2],
                gbsems[c % 2])

        def w(c):
            return pltpu.make_async_copy(
                abufs[c % 2], out_hbm.at[pl.ds(base + c * ch, ch)],
                wsems[c % 2])

        ga(0).start(); gb(0).start()
        ga(1).start(); gb(1).start()
        for c in range(nch):
            s = c % 2
            ga(c).wait()
            gb(c).wait()
            ba, bc = abufs[s], cbufs[s]

            @pl.loop(0, ch)
            def _row(r):
                for j in range(zdim // 16):
                    sl = pl.ds(j * 16, 16)
                    ba[r, sl] = ba[r, sl] + bc[r, sl]

            w(c).start()
            if c + 2 < nch:
                w(c).wait()
                ga(c + 2).start()
                gb(c + 2).start()
        w(nch - 2).wait()
        w(nch - 1).wait()

    return k(zs, pos0, pos1)


def _gate_kernel(x_ref, y_ref, gwx_ref, gwy_ref, gb_ref, idx_ref, w_ref,
                 xi_ref):
    xb = x_ref[...]
    u = lax.bitcast_convert_type(xb, jnp.uint32)
    r = (u + jnp.uint32(0x7FFF) + ((u >> 16) & jnp.uint32(1))) >> 16
    half = xb.shape[1] // 2
    packed = lax.bitcast_convert_type(
        r[:, :half] | (r[:, half:] << 16), jnp.int32)
    xi_ref[...] = packed
    logits = jnp.dot(xb, gwx_ref[...], preferred_element_type=jnp.float32)
    logits = logits + y_ref[...] * gwy_ref[...] + gb_ref[...]
    e = logits.shape[1]
    m = jnp.max(logits, axis=1, keepdims=True)
    ex = jnp.exp(logits - m)
    p = ex / jnp.sum(ex, axis=1, keepdims=True)
    cols = jax.lax.broadcasted_iota(jnp.int32, p.shape, 1)
    m0 = jnp.max(p, axis=1, keepdims=True)
    i0 = jnp.min(jnp.where(p == m0, cols, e), axis=1)
    pm = jnp.where(cols == i0[:, None], -1.0, p)
    m1 = jnp.max(pm, axis=1, keepdims=True)
    i1 = jnp.min(jnp.where(pm == m1, cols, e), axis=1)
    p0 = m0[:, 0]
    p1 = m1[:, 0]
    s = p0 + p1
    idx_ref[:, 0:1] = i0[:, None]
    idx_ref[:, 1:2] = i1[:, None]
    w_ref[:, 0:1] = (p0 / s)[:, None]
    w_ref[:, 1:2] = (p1 / s)[:, None]


def _gate(x, y, gate_W, gate_b):
    b, xdim = x.shape
    e = gate_W.shape[1]
    gwx = gate_W[:xdim]
    gwy = gate_W[xdim:]
    gb = gate_b.reshape(1, e)
    grid = (b // GATE_BM,)
    return pl.pallas_call(
        _gate_kernel,
        grid=grid,
        in_specs=[
            pl.BlockSpec((GATE_BM, xdim), lambda i: (i, 0)),
            pl.BlockSpec((GATE_BM, 1), lambda i: (i, 0)),
            pl.BlockSpec((xdim, e), lambda i: (0, 0)),
            pl.BlockSpec((1, e), lambda i: (0, 0)),
            pl.BlockSpec((1, e), lambda i: (0, 0)),
        ],
        out_specs=[
            pl.BlockSpec((GATE_BM, 2), lambda i: (i, 0)),
            pl.BlockSpec((GATE_BM, 2), lambda i: (i, 0)),
            pl.BlockSpec((GATE_BM, xdim // 2), lambda i: (i, 0)),
        ],
        out_shape=[
            jax.ShapeDtypeStruct((b, 2), jnp.int32),
            jax.ShapeDtypeStruct((b, 2), jnp.float32),
            jax.ShapeDtypeStruct((b, xdim // 2), jnp.int32),
        ],
    )(x, y.reshape(b, 1), gwx, gwy, gb)


def _mlp_kernel(be_ref, xs_ref, ry_ref, rw_ref, w1_ref, b1_ref,
                w2_ref, b2_ref, w3_ref, b3_ref, out_ref):
    half = xs_ref.shape[1]
    xdim = 2 * half
    ui = lax.bitcast_convert_type(xs_ref[...], jnp.uint32)
    xlo = lax.bitcast_convert_type(ui << 16, jnp.float32)
    xhi = lax.bitcast_convert_type(ui & jnp.uint32(0xFFFF0000), jnp.float32)
    w1 = w1_ref[0]
    h1 = jnp.dot(xlo.astype(jnp.bfloat16), w1[:half].astype(jnp.bfloat16),
                 preferred_element_type=jnp.float32)
    h1 = h1 + jnp.dot(xhi.astype(jnp.bfloat16),
                      w1[half:xdim].astype(jnp.bfloat16),
                      preferred_element_type=jnp.float32)
    h1 = h1 + ry_ref[...] * w1[xdim:xdim + 1] + b1_ref[0]
    h1 = jnp.maximum(h1, 0.0)
    h2 = jnp.dot(h1.astype(jnp.bfloat16), w2_ref[0].astype(jnp.bfloat16),
                 preferred_element_type=jnp.float32)
    h2 = jnp.maximum(h2 + b2_ref[0], 0.0)
    z = jnp.dot(h2.astype(jnp.bfloat16), w3_ref[0].astype(jnp.bfloat16),
                preferred_element_type=jnp.float32)
    out_ref[...] = (z + b3_ref[0]) * rw_ref[...]


def _grouped_mlp(xs, row_y, row_w, block_expert, W1, b1, W2, b2, W3, b3):
    p, xhalf = xs.shape
    e, d, h1 = W1.shape
    h2 = W2.shape[2]
    z1 = W3.shape[2]
    nb = p // BM
    grid_spec = pltpu.PrefetchScalarGridSpec(
        num_scalar_prefetch=1,
        grid=(nb,),
        in_specs=[
            pl.BlockSpec((BM, xhalf), lambda i, be: (i, 0)),
            pl.BlockSpec((BM, 1), lambda i, be: (i, 0)),
            pl.BlockSpec((BM, 1), lambda i, be: (i, 0)),
            pl.BlockSpec((1, d, h1), lambda i, be: (be[i], 0, 0)),
            pl.BlockSpec((1, 1, h1), lambda i, be: (be[i], 0, 0)),
            pl.BlockSpec((1, h1, h2), lambda i, be: (be[i], 0, 0)),
            pl.BlockSpec((1, 1, h2), lambda i, be: (be[i], 0, 0)),
            pl.BlockSpec((1, h2, z1), lambda i, be: (be[i], 0, 0)),
            pl.BlockSpec((1, 1, z1), lambda i, be: (be[i], 0, 0)),
        ],
        out_specs=pl.BlockSpec((BM, z1), lambda i, be: (i, 0)),
    )
    return pl.pallas_call(
        _mlp_kernel,
        grid_spec=grid_spec,
        out_shape=jax.ShapeDtypeStruct((p, z1), jnp.float32),
        compiler_params=pltpu.CompilerParams(
            dimension_semantics=("arbitrary",),
        ),
    )(block_expert, xs, row_y.reshape(p, 1), row_w.reshape(p, 1),
      W1, b1.reshape(e, 1, h1), W2, b2.reshape(e, 1, h2),
      W3, b3.reshape(e, 1, z1))


def kernel(x, y, gate_W, gate_b, W1, b1, W2, b2, W3, b3):
    b = x.shape[0]
    e = W1.shape[0]

    idx, w, x16 = _gate(x, y, gate_W, gate_b)

    ef = idx.reshape(-1)
    wf = w.reshape(-1)
    tf = jnp.repeat(jnp.arange(b, dtype=jnp.int32), 2)
    oh = (ef[:, None] == jnp.arange(e, dtype=jnp.int32)[None, :]).astype(jnp.int32)
    cum = jnp.cumsum(oh, axis=0)
    rank = jnp.take_along_axis(cum, ef[:, None], axis=1)[:, 0] - 1
    counts = cum[-1]
    padded = ((counts + BM - 1) // BM) * BM
    pad_end = jnp.cumsum(padded)
    pad_start = pad_end - padded
    dst = pad_start[ef] + rank
    nb = (2 * b + e * (BM - 1) + BM - 1) // BM
    p = nb * BM
    row_token = jnp.zeros((p,), jnp.int32).at[dst].set(tf)
    row_y = jnp.zeros((p,), jnp.float32).at[dst].set(jnp.repeat(y, 2))
    row_w = jnp.zeros((p,), jnp.float32).at[dst].set(wf)
    block_starts = jnp.arange(nb, dtype=jnp.int32) * BM
    block_expert = jnp.minimum(
        jnp.sum(block_starts[:, None] >= pad_end[None, :], axis=1), e - 1
    ).astype(jnp.int32)

    xs = _sc_gather(x16, row_token, p)

    zs = _grouped_mlp(xs, row_y, row_w, block_expert, W1, b1, W2, b2, W3, b3)

    pos = dst.reshape(b, 2)
    out = _sc_combine(zs, pos[:, 0], pos[:, 1])
    return out


# speedup vs baseline: 1.0477x; 1.0477x over previous
import functools

import jax
import jax.numpy as jnp
from jax import lax
from jax.experimental import pallas as pl
from jax.experimental.pallas import tpu as pltpu
from jax.experimental.pallas import tpu_sc as plsc

BM = 256
GATE_BM = 512
_NC = 2
_NS = 16
_NW = _NC * _NS


def _sc_mesh():
    return plsc.VectorSubcoreMesh(core_axis_name="c", subcore_axis_name="s")


def _sc_gather(x, row_token, p):
    row_shape = x.shape[1:]
    row_elems = 1
    for s in row_shape:
        row_elems *= s
    row_bytes = row_elems * x.dtype.itemsize
    rpw = p // _NW
    ch = max(8, (128 * 1024) // row_bytes)
    while rpw % ch:
        ch //= 2
    nbuf = 3
    nch = rpw // ch

    @functools.partial(
        pl.kernel,
        out_type=jax.ShapeDtypeStruct((p,) + row_shape, x.dtype),
        mesh=_sc_mesh(),
        scratch_types=[
            pltpu.VMEM((rpw,), jnp.int32),
            [pltpu.VMEM((ch,) + row_shape, x.dtype)] * nbuf,
            [pltpu.SemaphoreType.DMA] * nbuf,
            [pltpu.SemaphoreType.DMA] * nbuf,
        ],
    )
    def k(tok_hbm, x_hbm, xs_hbm, idx_v, bufs, gsems, wsems):
        wid = lax.axis_index("s") * _NC + lax.axis_index("c")
        base = wid * rpw
        pltpu.sync_copy(tok_hbm.at[pl.ds(base, rpw)], idx_v)

        def g(c):
            return pltpu.make_async_copy(
                x_hbm.at[idx_v.at[pl.ds(c * ch, ch)]], bufs[c % nbuf],
                gsems[c % nbuf])

        def w(c):
            return pltpu.make_async_copy(
                bufs[c % nbuf], xs_hbm.at[pl.ds(base + c * ch, ch)],
                wsems[c % nbuf])

        g(0).start()
        g(1).start()
        for c in range(nch):
            g(c).wait()
            w(c).start()
            if c + 2 < nch:
                if c >= 1:
                    w(c - 1).wait()
                g(c + 2).start()
        for c in range(max(nch - 3, 0), nch):
            w(c).wait()

    return k(row_token, x)


def _sc_combine(zs, pos0, pos1):
    b = pos0.shape[0]
    zdim = zs.shape[1]
    tpw = b // _NW
    ch = 64
    nch = tpw // ch

    @functools.partial(
        pl.kernel,
        out_type=jax.ShapeDtypeStruct((b, zdim), jnp.float32),
        mesh=_sc_mesh(),
        scratch_types=[
            pltpu.VMEM((tpw,), jnp.int32),
            pltpu.VMEM((tpw,), jnp.int32),
            pltpu.VMEM((ch, zdim), jnp.float32),
            pltpu.VMEM((ch, zdim), jnp.float32),
            pltpu.VMEM((ch, zdim), jnp.float32),
            pltpu.VMEM((ch, zdim), jnp.float32),
            pltpu.SemaphoreType.DMA,
            pltpu.SemaphoreType.DMA,
            pltpu.SemaphoreType.DMA,
            pltpu.SemaphoreType.DMA,
            pltpu.SemaphoreType.DMA,
            pltpu.SemaphoreType.DMA,
        ],
    )
    def k(zs_hbm, p0_hbm, p1_hbm, out_hbm, i0_v, i1_v,
          a0, a1, c0, c1, ga0, ga1, gb0, gb1, w0, w1):
        wid = lax.axis_index("s") * _NC + lax.axis_index("c")
        base = wid * tpw
        pltpu.sync_copy(p0_hbm.at[pl.ds(base, tpw)], i0_v)
        pltpu.sync_copy(p1_hbm.at[pl.ds(base, tpw)], i1_v)
        abufs, cbufs = (a0, a1), (c0, c1)
        gasems, gbsems, wsems = (ga0, ga1), (gb0, gb1), (w0, w1)

        def ga(c):
            return pltpu.make_async_copy(
                zs_hbm.at[i0_v.at[pl.ds(c * ch, ch)]], abufs[c % 2],
                gasems[c % 2])

        def gb(c):
            return pltpu.make_async_copy(
                zs_hbm.at[i1_v.at[pl.ds(c * ch, ch)]], cbufs[c % 2],
                gbsems[c % 2])

        def w(c):
            return pltpu.make_async_copy(
                abufs[c % 2], out_hbm.at[pl.ds(base + c * ch, ch)],
                wsems[c % 2])

        ga(0).start(); gb(0).start()
        ga(1).start(); gb(1).start()
        for c in range(nch):
            s = c % 2
            ga(c).wait()
            gb(c).wait()
            ba, bc = abufs[s], cbufs[s]

            @pl.loop(0, ch)
            def _row(r):
                for j in range(zdim // 16):
                    sl = pl.ds(j * 16, 16)
                    ba[r, sl] = ba[r, sl] + bc[r, sl]

            w(c).start()
            if c + 2 < nch:
                w(c).wait()
                ga(c + 2).start()
                gb(c + 2).start()
        w(nch - 2).wait()
        w(nch - 1).wait()

    return k(zs, pos0, pos1)


def _gate_kernel(x_ref, y_ref, gwx_ref, gwy_ref, gb_ref, idx_ref, w_ref,
                 xi_ref):
    xb = x_ref[...]
    u = lax.bitcast_convert_type(xb, jnp.uint32)
    r = (u + jnp.uint32(0x7FFF) + ((u >> 16) & jnp.uint32(1))) >> 16
    half = xb.shape[1] // 2
    packed = lax.bitcast_convert_type(
        r[:, :half] | (r[:, half:] << 16), jnp.int32)
    xi_ref[...] = packed
    logits = jnp.dot(xb, gwx_ref[...], preferred_element_type=jnp.float32)
    logits = logits + y_ref[...] * gwy_ref[...] + gb_ref[...]
    e = logits.shape[1]
    m = jnp.max(logits, axis=1, keepdims=True)
    ex = jnp.exp(logits - m)
    p = ex / jnp.sum(ex, axis=1, keepdims=True)
    cols = jax.lax.broadcasted_iota(jnp.int32, p.shape, 1)
    m0 = jnp.max(p, axis=1, keepdims=True)
    i0 = jnp.min(jnp.where(p == m0, cols, e), axis=1)
    pm = jnp.where(cols == i0[:, None], -1.0, p)
    m1 = jnp.max(pm, axis=1, keepdims=True)
    i1 = jnp.min(jnp.where(pm == m1, cols, e), axis=1)
    p0 = m0[:, 0]
    p1 = m1[:, 0]
    s = p0 + p1
    idx_ref[:, 0:1] = i0[:, None]
    idx_ref[:, 1:2] = i1[:, None]
    w_ref[:, 0:1] = (p0 / s)[:, None]
    w_ref[:, 1:2] = (p1 / s)[:, None]


def _gate(x, y, gate_W, gate_b):
    b, xdim = x.shape
    e = gate_W.shape[1]
    gwx = gate_W[:xdim]
    gwy = gate_W[xdim:]
    gb = gate_b.reshape(1, e)
    grid = (b // GATE_BM,)
    return pl.pallas_call(
        _gate_kernel,
        grid=grid,
        in_specs=[
            pl.BlockSpec((GATE_BM, xdim), lambda i: (i, 0)),
            pl.BlockSpec((GATE_BM, 1), lambda i: (i, 0)),
            pl.BlockSpec((xdim, e), lambda i: (0, 0)),
            pl.BlockSpec((1, e), lambda i: (0, 0)),
            pl.BlockSpec((1, e), lambda i: (0, 0)),
        ],
        out_specs=[
            pl.BlockSpec((GATE_BM, 2), lambda i: (i, 0)),
            pl.BlockSpec((GATE_BM, 2), lambda i: (i, 0)),
            pl.BlockSpec((GATE_BM, xdim // 2), lambda i: (i, 0)),
        ],
        out_shape=[
            jax.ShapeDtypeStruct((b, 2), jnp.int32),
            jax.ShapeDtypeStruct((b, 2), jnp.float32),
            jax.ShapeDtypeStruct((b, xdim // 2), jnp.int32),
        ],
    )(x, y.reshape(b, 1), gwx, gwy, gb)


def _mlp_kernel(be_ref, xs_ref, ry_ref, rw_ref, w1_ref, b1_ref,
                w2_ref, b2_ref, w3_ref, b3_ref, out_ref):
    half = xs_ref.shape[1]
    xdim = 2 * half
    ui = lax.bitcast_convert_type(xs_ref[...], jnp.uint32)
    xlo = lax.bitcast_convert_type(ui << 16, jnp.float32)
    xhi = lax.bitcast_convert_type(ui & jnp.uint32(0xFFFF0000), jnp.float32)
    w1 = w1_ref[0]
    h1 = jnp.dot(xlo.astype(jnp.bfloat16), w1[:half].astype(jnp.bfloat16),
                 preferred_element_type=jnp.float32)
    h1 = h1 + jnp.dot(xhi.astype(jnp.bfloat16),
                      w1[half:xdim].astype(jnp.bfloat16),
                      preferred_element_type=jnp.float32)
    h1 = h1 + ry_ref[...] * w1[xdim:xdim + 1] + b1_ref[0]
    h1 = jnp.maximum(h1, 0.0)
    h2 = jnp.dot(h1.astype(jnp.bfloat16), w2_ref[0].astype(jnp.bfloat16),
                 preferred_element_type=jnp.float32)
    h2 = jnp.maximum(h2 + b2_ref[0], 0.0)
    z = jnp.dot(h2.astype(jnp.bfloat16), w3_ref[0].astype(jnp.bfloat16),
                preferred_element_type=jnp.float32)
    out_ref[...] = (z + b3_ref[0]) * rw_ref[...]


def _grouped_mlp(xs, row_y, row_w, block_expert, W1, b1, W2, b2, W3, b3):
    p, xhalf = xs.shape
    e, d, h1 = W1.shape
    h2 = W2.shape[2]
    z1 = W3.shape[2]
    nb = p // BM
    grid_spec = pltpu.PrefetchScalarGridSpec(
        num_scalar_prefetch=1,
        grid=(nb,),
        in_specs=[
            pl.BlockSpec((BM, xhalf), lambda i, be: (i, 0)),
            pl.BlockSpec((BM, 1), lambda i, be: (i, 0)),
            pl.BlockSpec((BM, 1), lambda i, be: (i, 0)),
            pl.BlockSpec((1, d, h1), lambda i, be: (be[i], 0, 0)),
            pl.BlockSpec((1, 1, h1), lambda i, be: (be[i], 0, 0)),
            pl.BlockSpec((1, h1, h2), lambda i, be: (be[i], 0, 0)),
            pl.BlockSpec((1, 1, h2), lambda i, be: (be[i], 0, 0)),
            pl.BlockSpec((1, h2, z1), lambda i, be: (be[i], 0, 0)),
            pl.BlockSpec((1, 1, z1), lambda i, be: (be[i], 0, 0)),
        ],
        out_specs=pl.BlockSpec((BM, z1), lambda i, be: (i, 0)),
    )
    return pl.pallas_call(
        _mlp_kernel,
        grid_spec=grid_spec,
        out_shape=jax.ShapeDtypeStruct((p, z1), jnp.float32),
        compiler_params=pltpu.CompilerParams(
            dimension_semantics=("arbitrary",),
        ),
    )(block_expert, xs, row_y.reshape(p, 1), row_w.reshape(p, 1),
      W1, b1.reshape(e, 1, h1), W2, b2.reshape(e, 1, h2),
      W3, b3.reshape(e, 1, z1))


def kernel(x, y, gate_W, gate_b, W1, b1, W2, b2, W3, b3):
    b = x.shape[0]
    e = W1.shape[0]

    idx, w, x16 = _gate(x, y, gate_W, gate_b)

    ef = idx.reshape(-1)
    wf = w.reshape(-1)
    tf = jnp.repeat(jnp.arange(b, dtype=jnp.int32), 2)
    oh = (ef[:, None] == jnp.arange(e, dtype=jnp.int32)[None, :]).astype(jnp.int32)
    cum = jnp.cumsum(oh, axis=0)
    rank = jnp.take_along_axis(cum, ef[:, None], axis=1)[:, 0] - 1
    counts = cum[-1]
    padded = ((counts + BM - 1) // BM) * BM
    pad_end = jnp.cumsum(padded)
    pad_start = pad_end - padded
    dst = pad_start[ef] + rank
    nb = (2 * b + e * (BM - 1) + BM - 1) // BM
    p = nb * BM
    row_token = jnp.zeros((p,), jnp.int32).at[dst].set(tf)
    row_y = jnp.zeros((p,), jnp.float32).at[dst].set(jnp.repeat(y, 2))
    row_w = jnp.zeros((p,), jnp.float32).at[dst].set(wf)
    block_starts = jnp.arange(nb, dtype=jnp.int32) * BM
    block_expert = jnp.minimum(
        jnp.sum(block_starts[:, None] >= pad_end[None, :], axis=1), e - 1
    ).astype(jnp.int32)

    xs = _sc_gather(x16, row_token, p)

    zs = _grouped_mlp(xs, row_y, row_w, block_expert, W1, b1, W2, b2, W3, b3)

    pos = dst.reshape(b, 2)
    out = _sc_combine(zs, pos[:, 0], pos[:, 1])
    return out
